# baseline (speedup 1.0000x reference)
import functools

import jax
import jax.numpy as jnp
from jax import lax
from jax.experimental import pallas as pl
from jax.experimental.pallas import tpu as pltpu
from jax.experimental.pallas import tpu_sc as plsc

_NC = 2
_NS = 16
_NW = _NC * _NS


def _row_chunks(total, ch):
    out = [ch] * (total // ch)
    if total % ch:
        out.append(total % ch)
    return out


def _pre_body(x_ref, wpre_ref, bpre_ref, wd_ref, ws_ref, h_ref, d_ref, s_ref):
    h = jnp.maximum(x_ref[...] @ wpre_ref[...] + bpre_ref[...], 0.0)
    h_ref[...] = h
    d_ref[...] = h @ wd_ref[...]
    s_ref[...] = h @ ws_ref[...]


def _inv_body(cnt_ref, inv_ref):
    cs = jnp.sum(cnt_ref[...], axis=0)
    inv_ref[...] = (1.0 / jnp.maximum(cs, 1.0))[:, None]


def _mid_body(h_ref, agg_ref, inv_ref, wd_ref, ws_ref, hn_ref, d_ref, s_ref):
    inv = inv_ref[...]
    hn = jnp.maximum(h_ref[...] + (agg_ref[0] + agg_ref[1]) * inv, 0.0)
    hn_ref[...] = hn
    d_ref[...] = hn @ wd_ref[...]
    s_ref[...] = hn @ ws_ref[...]


def _edgemm_body(ea_ref, w_ref, b_ref, c_ref):
    c_ref[...] = ea_ref[...] @ w_ref[...] + b_ref[...]


def _post_body(h_ref, agg_ref, inv_ref, bm_ref, wpost_ref, bpost_ref,
               wout_ref, bout_ref, out_ref):
    n, _ = h_ref.shape
    hn = jnp.maximum(h_ref[...] + (agg_ref[0] + agg_ref[1]) * inv_ref[...], 0.0)
    nb = out_ref.shape[0]
    onehot = (bm_ref[...] == lax.broadcasted_iota(jnp.int32, (n, nb), 1))
    onehot = onehot.astype(jnp.float32)
    psum = lax.dot_general(onehot, hn, (((0,), (0,)), ((), ())))
    pcnt = jnp.sum(onehot, axis=0)
    pooled = psum * (1.0 / jnp.maximum(pcnt, 1.0))[:, None]
    r = jnp.maximum(pooled @ wpost_ref[...] + bpost_ref[...], 0.0)
    out_ref[...] = r @ wout_ref[...] + bout_ref[...]


def _make_cnt_kernel(e, n):
    npad = ((n + 127) // 128) * 128
    epw = e // _NW
    chn = 2000
    assert epw % chn == 0 and chn % 16 == 0
    mesh = plsc.VectorSubcoreMesh(core_axis_name="c", subcore_axis_name="s")

    @functools.partial(
        pl.kernel,
        out_type=jax.ShapeDtypeStruct((_NW, npad), jnp.float32),
        mesh=mesh,
        compiler_params=pltpu.CompilerParams(needs_layout_passes=False),
        scratch_types=[
            pltpu.VMEM((npad,), jnp.float32),
            pltpu.VMEM((chn,), jnp.int32),
        ],
    )
    def cnt_kernel(dst_hbm, out_hbm, cnt_v, idx_v):
        cc = lax.axis_index("c")
        ss = lax.axis_index("s")
        wid = ss * _NC + cc

        def zbody(i, carry):
            cnt_v[pl.ds(i * 16, 16)] = jnp.zeros((16,), jnp.float32)
            return carry

        lax.fori_loop(0, npad // 16, zbody, 0)
        base = wid * epw
        ones = jnp.ones((16,), jnp.float32)
        for ck in range(epw // chn):
            pltpu.sync_copy(dst_hbm.at[pl.ds(base + ck * chn, chn)], idx_v)

            def abody(t, carry):
                iv = idx_v[pl.ds(t * 16, 16)]
                plsc.addupdate_scatter(cnt_v, [iv], ones)
                return carry

            lax.fori_loop(0, chn // 16, abody, 0)
        pltpu.sync_copy(cnt_v, out_hbm.at[wid])

    return cnt_kernel


def _make_edge_kernel(e, n, f):
    epw = e // _NW
    ch = 40
    assert epw % ch == 0 and ch % 8 == 0
    n2 = _NS * ((n + _NS * 8 - 1) // (_NS * 8)) * 8
    rpt = n2 // _NS
    f2 = 2 * f
    mesh = plsc.VectorSubcoreMesh(core_axis_name="c", subcore_axis_name="s")

    @functools.partial(
        pl.kernel,
        out_type=jax.ShapeDtypeStruct((_NC, n2, f), jnp.float32),
        mesh=mesh,
        compiler_params=pltpu.CompilerParams(needs_layout_passes=False),
        scratch_types=[
            pltpu.VMEM((ch,), jnp.int32),
            pltpu.VMEM((ch,), jnp.int32),
            pltpu.VMEM((ch, f2), jnp.float32),
            pltpu.VMEM((ch, f2), jnp.float32),
            pltpu.VMEM((ch, f2), jnp.float32),
            pltpu.VMEM((ch, f), jnp.float32),
            pltpu.VMEM_SHARED((n2, f), jnp.float32),
            pltpu.SemaphoreType.DMA,
            pltpu.SemaphoreType.DMA,
            pltpu.SemaphoreType.DMA,
        ],
    )
    def edge_kernel(dst_hbm, src_hbm, d_hbm, s_hbm, c_hbm, out_hbm,
                    idx_d, idx_s, drows, srows, crows, m_v, agg_sh,
                    semd, sems, semc):
        cc = lax.axis_index("c")
        ss = lax.axis_index("s")
        wid = ss * _NC + cc

        def zbody(ee, carry):
            for j in range(f // 16):
                m_v[ee, pl.ds(j * 16, 16)] = jnp.zeros((16,), jnp.float32)
            return carry

        lax.fori_loop(0, ch, zbody, 0)
        r0 = ss * rpt
        off = 0
        for nr in _row_chunks(rpt, ch):
            pltpu.sync_copy(m_v.at[pl.ds(0, nr)], agg_sh.at[pl.ds(r0 + off, nr)])
            off += nr
        plsc.subcore_barrier()

        base = wid * epw
        c3, c5, c7, c9 = 1.0 / 3.0, 1.0 / 5.0, 1.0 / 7.0, 1.0 / 9.0

        def chunk(k, carry):
            b0 = base + k * ch
            pltpu.sync_copy(dst_hbm.at[pl.ds(b0, ch)], idx_d)
            pltpu.sync_copy(src_hbm.at[pl.ds(b0, ch)], idx_s)
            cpd = pltpu.async_copy(d_hbm.at[idx_d], drows, semd)
            cps = pltpu.async_copy(s_hbm.at[idx_s], srows, sems)
            cpc = pltpu.async_copy(c_hbm.at[pl.ds(b0, ch)], crows, semc)
            cpd.wait()
            cps.wait()
            cpc.wait()

            def ebody(ee, carry2):
                for j in range(f // 16):
                    o = j * 16
                    zf = (drows[ee, pl.ds(o, 16)] + srows[ee, pl.ds(o, 16)]
                          + crows[ee, pl.ds(o, 16)])
                    zs = (drows[ee, pl.ds(f + o, 16)] + srows[ee, pl.ds(f + o, 16)]
                          + crows[ee, pl.ds(f + o, 16)])
                    sig = 1.0 / (1.0 + jnp.exp(-zf))
                    u = jnp.exp(-jnp.abs(zs))
                    w = u / (2.0 + u)
                    w2 = w * w
                    sp = jnp.maximum(zs, 0.0) + 2.0 * w * (
                        1.0 + w2 * (c3 + w2 * (c5 + w2 * (c7 + w2 * c9))))
                    m_v[ee, pl.ds(o, 16)] = sig * sp
                return carry2

            lax.fori_loop(0, ch, ebody, 0)
            pltpu.sync_copy(m_v, agg_sh.at[idx_d], add=True)
            return carry

        lax.fori_loop(0, epw // ch, chunk, 0)
        plsc.subcore_barrier()

        off = 0
        for nr in _row_chunks(rpt, ch):
            pltpu.sync_copy(agg_sh.at[pl.ds(r0 + off, nr)], m_v.at[pl.ds(0, nr)])
            pltpu.sync_copy(m_v.at[pl.ds(0, nr)],
                            out_hbm.at[cc, pl.ds(r0 + off, nr)])
            off += nr

    return edge_kernel


def kernel(X, edge_idx, edge_attr, batch_map, W_pre, b_pre, Wf, bf, Ws, bs,
           W_post, b_post, W_out, b_out):
    n, f = X.shape
    e = edge_idx.shape[1]
    fe = edge_attr.shape[1]
    ng = Wf.shape[0]
    b_graphs = 64
    f2 = 2 * f
    src = edge_idx[0]
    dst = edge_idx[1]

    wd = [jnp.concatenate([Wf[i, :f], Ws[i, :f]], axis=1) for i in range(ng)]
    wsrc = [jnp.concatenate([Wf[i, f:f2], Ws[i, f:f2]], axis=1) for i in range(ng)]
    we = [jnp.concatenate([Wf[i, f2:], Ws[i, f2:]], axis=1) for i in range(ng)]
    bc = [jnp.concatenate([bf[i], bs[i]])[None, :] for i in range(ng)]

    blk = 2000
    grid_n = n // blk
    eblk = 8000
    grid_e = e // eblk
    npad = ((n + 127) // 128) * 128

    pre_call = pl.pallas_call(
        _pre_body,
        grid=(grid_n,),
        in_specs=[
            pl.BlockSpec((blk, f), lambda i: (i, 0)),
            pl.BlockSpec((f, f), lambda i: (0, 0)),
            pl.BlockSpec((1, f), lambda i: (0, 0)),
            pl.BlockSpec((f, f2), lambda i: (0, 0)),
            pl.BlockSpec((f, f2), lambda i: (0, 0)),
        ],
        out_specs=[
            pl.BlockSpec((blk, f), lambda i: (i, 0)),
            pl.BlockSpec((blk, f2), lambda i: (i, 0)),
            pl.BlockSpec((blk, f2), lambda i: (i, 0)),
        ],
        out_shape=[
            jax.ShapeDtypeStruct((n, f), jnp.float32),
            jax.ShapeDtypeStruct((n, f2), jnp.float32),
            jax.ShapeDtypeStruct((n, f2), jnp.float32),
        ],
    )

    mid_call = pl.pallas_call(
        _mid_body,
        grid=(grid_n,),
        in_specs=[
            pl.BlockSpec((blk, f), lambda i: (i, 0)),
            pl.BlockSpec((2, blk, f), lambda i: (0, i, 0)),
            pl.BlockSpec((blk, 1), lambda i: (i, 0)),
            pl.BlockSpec((f, f2), lambda i: (0, 0)),
            pl.BlockSpec((f, f2), lambda i: (0, 0)),
        ],
        out_specs=[
            pl.BlockSpec((blk, f), lambda i: (i, 0)),
            pl.BlockSpec((blk, f2), lambda i: (i, 0)),
            pl.BlockSpec((blk, f2), lambda i: (i, 0)),
        ],
        out_shape=[
            jax.ShapeDtypeStruct((n, f), jnp.float32),
            jax.ShapeDtypeStruct((n, f2), jnp.float32),
            jax.ShapeDtypeStruct((n, f2), jnp.float32),
        ],
    )

    edgemm_call = pl.pallas_call(
        _edgemm_body,
        grid=(grid_e,),
        in_specs=[
            pl.BlockSpec((eblk, fe), lambda i: (i, 0)),
            pl.BlockSpec((fe, f2), lambda i: (0, 0)),
            pl.BlockSpec((1, f2), lambda i: (0, 0)),
        ],
        out_specs=pl.BlockSpec((eblk, f2), lambda i: (i, 0)),
        out_shape=jax.ShapeDtypeStruct((e, f2), jnp.float32),
    )

    post_call = pl.pallas_call(
        _post_body,
        grid=(1,),
        in_specs=[
            pl.BlockSpec((n, f), lambda i: (0, 0)),
            pl.BlockSpec((2, n, f), lambda i: (0, 0, 0)),
            pl.BlockSpec((n, 1), lambda i: (0, 0)),
            pl.BlockSpec((n, 1), lambda i: (0, 0)),
            pl.BlockSpec((f, f), lambda i: (0, 0)),
            pl.BlockSpec((1, f), lambda i: (0, 0)),
            pl.BlockSpec((f, 1), lambda i: (0, 0)),
            pl.BlockSpec((1, 1), lambda i: (0, 0)),
        ],
        out_specs=pl.BlockSpec((b_graphs, 1), lambda i: (0, 0)),
        out_shape=jax.ShapeDtypeStruct((b_graphs, 1), jnp.float32),
    )

    inv_call = pl.pallas_call(
        _inv_body,
        in_specs=[pl.BlockSpec((_NW, npad), lambda: (0, 0))],
        out_specs=pl.BlockSpec((npad, 1), lambda: (0, 0)),
        out_shape=jax.ShapeDtypeStruct((npad, 1), jnp.float32),
    )

    cnt_call = _make_cnt_kernel(e, n)
    edge_call = _make_edge_kernel(e, n, f)

    h, d_arr, s_arr = pre_call(X, W_pre, b_pre[None], wd[0], wsrc[0])
    cnt = cnt_call(dst)
    inv = inv_call(cnt)[:n]

    out = None
    for i in range(ng):
        c_arr = edgemm_call(edge_attr, we[i], bc[i])
        agg = edge_call(dst, src, d_arr, s_arr, c_arr)
        if i < ng - 1:
            h, d_arr, s_arr = mid_call(h, agg, inv, wd[i + 1], wsrc[i + 1])
        else:
            out = post_call(h, agg, inv, batch_map[:, None], W_post,
                            b_post[None], W_out, b_out[None])
    return out


# speedup vs baseline: 1.1852x; 1.1852x over previous
import functools

import jax
import jax.numpy as jnp
from jax import lax
from jax.experimental import pallas as pl
from jax.experimental.pallas import tpu as pltpu
from jax.experimental.pallas import tpu_sc as plsc

_NC = 2
_NS = 16
_NW = _NC * _NS

_LP1 = 0.999995088607432
_LP2 = -0.4998478066857295
_LP3 = 0.3316139067728619
_LP4 = -0.24010052625106307
_LP5 = 0.16648205096376367
_LP6 = -0.09413390867454136
_LP7 = 0.035458822713676605
_LP8 = -0.00632052254022418


def _softplus_num(zs):
    u = jnp.exp(-jnp.abs(zs))
    p = _LP8
    for c in (_LP7, _LP6, _LP5, _LP4, _LP3, _LP2, _LP1):
        p = p * u + c
    return jnp.maximum(zs, 0.0) + p * u


def _row_chunks(total, ch):
    out = [ch] * (total // ch)
    if total % ch:
        out.append(total % ch)
    return out


def _pre_body(x_ref, wpre_ref, bpre_ref, wd_ref, ws_ref, h_ref, d_ref, s_ref):
    h = jnp.maximum(x_ref[...] @ wpre_ref[...] + bpre_ref[...], 0.0)
    h_ref[...] = h
    d_ref[...] = h @ wd_ref[...]
    s_ref[...] = h @ ws_ref[...]


def _inv_body(cnt_ref, inv_ref):
    cs = jnp.sum(cnt_ref[...], axis=0)
    inv_ref[...] = (1.0 / jnp.maximum(cs, 1.0))[:, None]


def _mid_body(h_ref, agg_ref, inv_ref, wd_ref, ws_ref, hn_ref, d_ref, s_ref):
    hn = jnp.maximum(
        h_ref[...] + (agg_ref[0] + agg_ref[1]) * inv_ref[...], 0.0)
    hn_ref[...] = hn
    d_ref[...] = hn @ wd_ref[...]
    s_ref[...] = hn @ ws_ref[...]


def _edgemm_body(ea_ref, w_ref, b_ref, c_ref):
    c_ref[...] = ea_ref[...] @ w_ref[...] + b_ref[...]


def _post_body(h_ref, agg_ref, inv_ref, bm_ref, wpost_ref, bpost_ref,
               wout_ref, bout_ref, out_ref):
    n, _ = h_ref.shape
    hn = jnp.maximum(
        h_ref[...] + (agg_ref[0] + agg_ref[1]) * inv_ref[...], 0.0)
    nb = out_ref.shape[0]
    onehot = (bm_ref[...] == lax.broadcasted_iota(jnp.int32, (n, nb), 1))
    onehot = onehot.astype(jnp.float32)
    psum = lax.dot_general(onehot, hn, (((0,), (0,)), ((), ())))
    pcnt = jnp.sum(onehot, axis=0)
    pooled = psum * (1.0 / jnp.maximum(pcnt, 1.0))[:, None]
    r = jnp.maximum(pooled @ wpost_ref[...] + bpost_ref[...], 0.0)
    out_ref[...] = r @ wout_ref[...] + bout_ref[...]


def _make_cnt_kernel(e, n):
    npad = ((n + 127) // 128) * 128
    epw = e // _NW
    chn = 2000
    assert epw % chn == 0 and chn % 16 == 0
    mesh = plsc.VectorSubcoreMesh(core_axis_name="c", subcore_axis_name="s")

    @functools.partial(
        pl.kernel,
        out_type=jax.ShapeDtypeStruct((_NW, npad), jnp.float32),
        mesh=mesh,
        compiler_params=pltpu.CompilerParams(needs_layout_passes=False),
        scratch_types=[
            pltpu.VMEM((npad,), jnp.float32),
            pltpu.VMEM((chn,), jnp.int32),
        ],
    )
    def cnt_kernel(dst_hbm, out_hbm, cnt_v, idx_v):
        cc = lax.axis_index("c")
        ss = lax.axis_index("s")
        wid = ss * _NC + cc

        def zbody(i, carry):
            cnt_v[pl.ds(i * 16, 16)] = jnp.zeros((16,), jnp.float32)
            return carry

        lax.fori_loop(0, npad // 16, zbody, 0)
        base = wid * epw
        ones = jnp.ones((16,), jnp.float32)
        for ck in range(epw // chn):
            pltpu.sync_copy(dst_hbm.at[pl.ds(base + ck * chn, chn)], idx_v)

            def abody(t, carry):
                iv = idx_v[pl.ds(t * 16, 16)]
                plsc.addupdate_scatter(cnt_v, [iv], ones)
                return carry

            lax.fori_loop(0, chn // 16, abody, 0)
        pltpu.sync_copy(cnt_v, out_hbm.at[wid])

    return cnt_kernel


def _make_edge_kernel(e, n, f):
    epw = e // _NW
    ch = 40
    assert epw % ch == 0 and ch % 8 == 0
    nch = epw // ch
    n2 = _NS * ((n + _NS * 8 - 1) // (_NS * 8)) * 8
    rpt = n2 // _NS
    f2 = 2 * f
    mesh = plsc.VectorSubcoreMesh(core_axis_name="c", subcore_axis_name="s")

    @functools.partial(
        pl.kernel,
        out_type=jax.ShapeDtypeStruct((_NC, n2, f), jnp.float32),
        mesh=mesh,
        compiler_params=pltpu.CompilerParams(needs_layout_passes=False),
        scratch_types=[
            pltpu.VMEM((ch,), jnp.int32),
            pltpu.VMEM((ch,), jnp.int32),
            pltpu.VMEM((ch, f2), jnp.float32),
            pltpu.VMEM((ch, f2), jnp.float32),
            pltpu.VMEM((ch, f2), jnp.float32),
            pltpu.VMEM((ch, f), jnp.float32),
            pltpu.VMEM_SHARED((n2, f), jnp.float32),
            pltpu.SemaphoreType.DMA,
            pltpu.SemaphoreType.DMA,
            pltpu.SemaphoreType.DMA,
        ],
    )
    def edge_kernel(dst_hbm, src_hbm, d_hbm, s_hbm, c_hbm, out_hbm,
                    idx_d, idx_s, drows, srows, crows, m_v, agg_sh,
                    semd, sems, semc):
        cc = lax.axis_index("c")
        ss = lax.axis_index("s")
        wid = ss * _NC + cc

        def zbody(ee, carry):
            for j in range(f // 16):
                m_v[ee, pl.ds(j * 16, 16)] = jnp.zeros((16,), jnp.float32)
            return carry

        lax.fori_loop(0, ch, zbody, 0)
        r0 = ss * rpt
        off = 0
        for nr in _row_chunks(rpt, ch):
            pltpu.sync_copy(m_v.at[pl.ds(0, nr)], agg_sh.at[pl.ds(r0 + off, nr)])
            off += nr
        plsc.subcore_barrier()

        base = wid * epw

        def chunk(k, carry):
            b0 = base + k * ch
            pltpu.sync_copy(dst_hbm.at[pl.ds(b0, ch)], idx_d)
            pltpu.sync_copy(src_hbm.at[pl.ds(b0, ch)], idx_s)
            cpd = pltpu.async_copy(d_hbm.at[idx_d], drows, semd)
            cps = pltpu.async_copy(s_hbm.at[idx_s], srows, sems)
            cpc = pltpu.async_copy(c_hbm.at[pl.ds(b0, ch)], crows, semc)
            cpd.wait()
            cps.wait()
            cpc.wait()

            def ebody(ee, carry2):
                for j in range(f // 16):
                    o = j * 16
                    zf = (drows[ee, pl.ds(o, 16)] + srows[ee, pl.ds(o, 16)]
                          + crows[ee, pl.ds(o, 16)])
                    zs = (drows[ee, pl.ds(f + o, 16)]
                          + srows[ee, pl.ds(f + o, 16)]
                          + crows[ee, pl.ds(f + o, 16)])
                    num = _softplus_num(zs)
                    m_v[ee, pl.ds(o, 16)] = num / (1.0 + jnp.exp(-zf))
                return carry2

            lax.fori_loop(0, ch, ebody, 0)
            pltpu.sync_copy(m_v, agg_sh.at[idx_d], add=True)
            return carry

        lax.fori_loop(0, nch, chunk, 0)
        plsc.subcore_barrier()

        off = 0
        for nr in _row_chunks(rpt, ch):
            pltpu.sync_copy(agg_sh.at[pl.ds(r0 + off, nr)], m_v.at[pl.ds(0, nr)])
            pltpu.sync_copy(m_v.at[pl.ds(0, nr)],
                            out_hbm.at[cc, pl.ds(r0 + off, nr)])
            off += nr

    return edge_kernel


def kernel(X, edge_idx, edge_attr, batch_map, W_pre, b_pre, Wf, bf, Ws, bs,
           W_post, b_post, W_out, b_out):
    n, f = X.shape
    e = edge_idx.shape[1]
    fe = edge_attr.shape[1]
    ng = Wf.shape[0]
    b_graphs = 64
    f2 = 2 * f
    src = edge_idx[0]
    dst = edge_idx[1]

    wd = [jnp.concatenate([Wf[i, :f], Ws[i, :f]], axis=1) for i in range(ng)]
    wsrc = [jnp.concatenate([Wf[i, f:f2], Ws[i, f:f2]], axis=1)
            for i in range(ng)]
    we = [jnp.concatenate([Wf[i, f2:], Ws[i, f2:]], axis=1) for i in range(ng)]
    bc = [jnp.concatenate([bf[i], bs[i]])[None, :] for i in range(ng)]

    blk = 2000
    grid_n = n // blk
    eblk = 8000
    grid_e = e // eblk
    npad = ((n + 127) // 128) * 128

    _w0 = lambda i: (0, 0)
    _row = lambda i: (i, 0)

    pre_call = pl.pallas_call(
        _pre_body,
        grid=(grid_n,),
        in_specs=[
            pl.BlockSpec((blk, f), _row),
            pl.BlockSpec((f, f), _w0),
            pl.BlockSpec((1, f), _w0),
            pl.BlockSpec((f, f2), _w0),
            pl.BlockSpec((f, f2), _w0),
        ],
        out_specs=[
            pl.BlockSpec((blk, f), _row),
            pl.BlockSpec((blk, f2), _row),
            pl.BlockSpec((blk, f2), _row),
        ],
        out_shape=[
            jax.ShapeDtypeStruct((n, f), jnp.float32),
            jax.ShapeDtypeStruct((n, f2), jnp.float32),
            jax.ShapeDtypeStruct((n, f2), jnp.float32),
        ],
    )

    mid_call = pl.pallas_call(
        _mid_body,
        grid=(grid_n,),
        in_specs=[
            pl.BlockSpec((blk, f), _row),
            pl.BlockSpec((2, blk, f), lambda i: (0, i, 0)),
            pl.BlockSpec((blk, 1), _row),
            pl.BlockSpec((f, f2), _w0),
            pl.BlockSpec((f, f2), _w0),
        ],
        out_specs=[
            pl.BlockSpec((blk, f), _row),
            pl.BlockSpec((blk, f2), _row),
            pl.BlockSpec((blk, f2), _row),
        ],
        out_shape=[
            jax.ShapeDtypeStruct((n, f), jnp.float32),
            jax.ShapeDtypeStruct((n, f2), jnp.float32),
            jax.ShapeDtypeStruct((n, f2), jnp.float32),
        ],
    )

    edgemm_call = pl.pallas_call(
        _edgemm_body,
        grid=(grid_e,),
        in_specs=[
            pl.BlockSpec((eblk, fe), _row),
            pl.BlockSpec((fe, f2), _w0),
            pl.BlockSpec((1, f2), _w0),
        ],
        out_specs=pl.BlockSpec((eblk, f2), _row),
        out_shape=jax.ShapeDtypeStruct((e, f2), jnp.float32),
    )

    inv_call = pl.pallas_call(
        _inv_body,
        in_specs=[pl.BlockSpec((_NW, npad), lambda: (0, 0))],
        out_specs=pl.BlockSpec((npad, 1), lambda: (0, 0)),
        out_shape=jax.ShapeDtypeStruct((npad, 1), jnp.float32),
    )

    post_call = pl.pallas_call(
        _post_body,
        grid=(1,),
        in_specs=[
            pl.BlockSpec((n, f), lambda i: (0, 0)),
            pl.BlockSpec((2, n, f), lambda i: (0, 0, 0)),
            pl.BlockSpec((n, 1), lambda i: (0, 0)),
            pl.BlockSpec((n, 1), lambda i: (0, 0)),
            pl.BlockSpec((f, f), lambda i: (0, 0)),
            pl.BlockSpec((1, f), lambda i: (0, 0)),
            pl.BlockSpec((f, 1), lambda i: (0, 0)),
            pl.BlockSpec((1, 1), lambda i: (0, 0)),
        ],
        out_specs=pl.BlockSpec((b_graphs, 1), lambda i: (0, 0)),
        out_shape=jax.ShapeDtypeStruct((b_graphs, 1), jnp.float32),
    )

    cnt_call = _make_cnt_kernel(e, n)
    edge_call = _make_edge_kernel(e, n, f)

    h, d_arr, s_arr = pre_call(X, W_pre, b_pre[None], wd[0], wsrc[0])
    cnt = cnt_call(dst)
    inv = inv_call(cnt)[:n]

    out = None
    for i in range(ng):
        c_arr = edgemm_call(edge_attr, we[i], bc[i])
        agg = edge_call(dst, src, d_arr, s_arr, c_arr)
        if i < ng - 1:
            h, d_arr, s_arr = mid_call(h, agg, inv, wd[i + 1], wsrc[i + 1])
        else:
            out = post_call(h, agg, inv, batch_map[:, None], W_post,
                            b_post[None], W_out, b_out[None])
    return out
